# Initial kernel scaffold; baseline (speedup 1.0000x reference)
import jax
import jax.numpy as jnp
from jax.experimental import pallas as pl


def kernel(ssl_content, proj_w, proj_b, codebook):
    raise NotImplementedError("write your pallas kernel here")


# speedup vs baseline: 1.0286x; 1.0286x over previous
import jax
import jax.numpy as jnp
from jax.experimental import pallas as pl


def _vq_kernel(ssl_ref, wt_ref, b_ref, cbt_ref, out_ref):
    ssl_blk = ssl_ref[0]
    xt = jax.lax.dot_general(
        ssl_blk, wt_ref[...],
        dimension_numbers=(((0,), (0,)), ((), ())),
        preferred_element_type=jnp.float32,
    )
    xt = xt + b_ref[...]
    scores = jnp.dot(xt, cbt_ref[...], preferred_element_type=jnp.float32)
    c2 = jnp.sum(cbt_ref[...] * cbt_ref[...], axis=0)[None, :]
    vals = c2 - 2.0 * scores
    out_ref[0, 0, :] = jnp.argmin(vals, axis=1).astype(jnp.int32)


@jax.jit
def kernel(ssl_content, proj_w, proj_b, codebook):
    B, C, T = ssl_content.shape
    K = codebook.shape[0]
    TB = 256
    n_tb = T // TB
    grid = (B * n_tb,)

    out = pl.pallas_call(
        _vq_kernel,
        grid=grid,
        in_specs=[
            pl.BlockSpec((1, C, TB), lambda i: (i // n_tb, 0, i % n_tb)),
            pl.BlockSpec((C, C), lambda i: (0, 0)),
            pl.BlockSpec((1, C), lambda i: (0, 0)),
            pl.BlockSpec((C, K), lambda i: (0, 0)),
        ],
        out_specs=pl.BlockSpec((1, 1, TB), lambda i: (i, 0, 0)),
        out_shape=jax.ShapeDtypeStruct((B * n_tb, 1, TB), jnp.int32),
    )(ssl_content, proj_w.T, proj_b.reshape(1, C), codebook.T)
    return out.reshape(B, T)


# speedup vs baseline: 1.0496x; 1.0204x over previous
import jax
import jax.numpy as jnp
from jax.experimental import pallas as pl
from jax.experimental.pallas import tpu as pltpu


def _c2_kernel(cbt_ref, c2_ref):
    c2_ref[...] = jnp.sum(cbt_ref[...] * cbt_ref[...], axis=0)[None, :]


def _vq_kernel(ssl_ref, wt_ref, b_ref, cbt_ref, c2_ref, out_ref):
    xt = jax.lax.dot_general(
        ssl_ref[0], wt_ref[...],
        dimension_numbers=(((0,), (0,)), ((), ())),
        preferred_element_type=jnp.float32,
    )
    xt = xt + b_ref[...]
    scores = jnp.dot(xt, cbt_ref[...], preferred_element_type=jnp.float32)
    vals = c2_ref[...] - 2.0 * scores
    out_ref[0, 0, :] = jnp.argmin(vals, axis=1).astype(jnp.int32)


@jax.jit
def kernel(ssl_content, proj_w, proj_b, codebook):
    B, C, T = ssl_content.shape
    K = codebook.shape[0]
    cbt = codebook.T

    c2 = pl.pallas_call(
        _c2_kernel,
        out_shape=jax.ShapeDtypeStruct((1, K), jnp.float32),
    )(cbt)

    TB = 512
    n_tb = T // TB
    out = pl.pallas_call(
        _vq_kernel,
        grid=(B * n_tb,),
        in_specs=[
            pl.BlockSpec((1, C, TB), lambda i: (i // n_tb, 0, i % n_tb)),
            pl.BlockSpec((C, C), lambda i: (0, 0)),
            pl.BlockSpec((1, C), lambda i: (0, 0)),
            pl.BlockSpec((C, K), lambda i: (0, 0)),
            pl.BlockSpec((1, K), lambda i: (0, 0)),
        ],
        out_specs=pl.BlockSpec((1, 1, TB), lambda i: (i, 0, 0)),
        out_shape=jax.ShapeDtypeStruct((B * n_tb, 1, TB), jnp.int32),
        compiler_params=pltpu.CompilerParams(
            dimension_semantics=("parallel",)),
    )(ssl_content, proj_w.T, proj_b.reshape(1, C), cbt, c2)
    return out.reshape(B, T)


# speedup vs baseline: 1.1412x; 1.0873x over previous
import jax
import jax.numpy as jnp
from jax.experimental import pallas as pl
from jax.experimental.pallas import tpu as pltpu

_HB = 256


def _c2_kernel(cbt_ref, c2_ref):
    c2_ref[...] = jnp.sum(cbt_ref[...] * cbt_ref[...], axis=0)[None, :]


def _vq_kernel(ssl_ref, wt_ref, b_ref, cbt_ref, c2_ref, out_ref):
    def scores_half(h):
        xt = jax.lax.dot_general(
            ssl_ref[0, :, h * _HB:(h + 1) * _HB], wt_ref[...],
            dimension_numbers=(((0,), (0,)), ((), ())),
            preferred_element_type=jnp.float32,
        )
        xt = xt + b_ref[...]
        return jnp.dot(xt, cbt_ref[...], preferred_element_type=jnp.float32)

    def amin_half(scores):
        vals = c2_ref[...] - 2.0 * scores
        return jnp.argmin(vals, axis=1).astype(jnp.int32)

    s_a = scores_half(0)
    s_b = scores_half(1)
    out_ref[0, 0, :_HB] = amin_half(s_a)
    out_ref[0, 0, _HB:] = amin_half(s_b)


@jax.jit
def kernel(ssl_content, proj_w, proj_b, codebook):
    B, C, T = ssl_content.shape
    K = codebook.shape[0]
    cbt = codebook.T

    c2 = pl.pallas_call(
        _c2_kernel,
        out_shape=jax.ShapeDtypeStruct((1, K), jnp.float32),
    )(cbt)

    TB = 512
    n_tb = T // TB
    out = pl.pallas_call(
        _vq_kernel,
        grid=(B * n_tb,),
        in_specs=[
            pl.BlockSpec((1, C, TB), lambda i: (i // n_tb, 0, i % n_tb)),
            pl.BlockSpec((C, C), lambda i: (0, 0)),
            pl.BlockSpec((1, C), lambda i: (0, 0)),
            pl.BlockSpec((C, K), lambda i: (0, 0)),
            pl.BlockSpec((1, K), lambda i: (0, 0)),
        ],
        out_specs=pl.BlockSpec((1, 1, TB), lambda i: (i, 0, 0)),
        out_shape=jax.ShapeDtypeStruct((B * n_tb, 1, TB), jnp.int32),
        compiler_params=pltpu.CompilerParams(
            dimension_semantics=("parallel",)),
    )(ssl_content, proj_w.T, proj_b.reshape(1, C), cbt, c2)
    return out.reshape(B, T)
